# baseline (speedup 1.0000x reference)
import functools

import jax
import jax.numpy as jnp
from jax import lax
from jax.experimental import pallas as pl
from jax.experimental.pallas import tpu as pltpu
from jax.experimental.pallas import tpu_sc as plsc

_NC = 2
_NS = 16
_CHUNK = 16
_NBUF = 8


@functools.lru_cache(maxsize=None)
def _build_gather(rows, row_len, V, D):
    B = rows * row_len
    NW = _NC * _NS
    assert B % (8 * NW) == 0
    b_per_w = B // NW
    assert row_len % b_per_w == 0
    C = min(_CHUNK, b_per_w)
    n_chunks = b_per_w // C
    assert b_per_w % C == 0

    mesh = plsc.VectorSubcoreMesh(core_axis_name="c", subcore_axis_name="s")

    @functools.partial(
        pl.kernel,
        mesh=mesh,
        out_type=jax.ShapeDtypeStruct((B, D), jnp.float32),
        scratch_types=(
            [pltpu.VMEM((b_per_w,), jnp.int32)]
            + [pltpu.VMEM((C, D), jnp.float32) for _ in range(_NBUF)]
            + [pltpu.SemaphoreType.DMA, pltpu.SemaphoreType.DMA]
        ),
    )
    def gather_kernel(idx_hbm, table_hbm, out_hbm, idx_v, *rest):
        bufs = rest[:_NBUF]
        gsem, osem = rest[_NBUF], rest[_NBUF + 1]
        wid = lax.axis_index("s") * _NC + lax.axis_index("c")
        base = wid * b_per_w
        row_len = idx_hbm.shape[1]
        pltpu.sync_copy(
            idx_hbm.at[base // row_len, pl.ds(base % row_len, b_per_w)],
            idx_v)

        def gather(c, buf):
            return pltpu.async_copy(
                table_hbm.at[idx_v.at[pl.ds(c * C, C)]], buf, gsem)

        def put(c, buf):
            return pltpu.async_copy(
                buf, out_hbm.at[pl.ds(base + c * C, C)], osem)

        g = [None] * n_chunks
        o = [None] * n_chunks
        waited = [False] * n_chunks
        for c in range(min(_NBUF, n_chunks)):
            g[c] = gather(c, bufs[c % _NBUF])
        for c in range(n_chunks):
            buf = bufs[c % _NBUF]
            g[c].wait()
            o[c] = put(c, buf)
            nxt = c + _NBUF - 1
            if nxt < n_chunks and nxt >= _NBUF:
                free_chunk = nxt - _NBUF
                o[free_chunk].wait()
                waited[free_chunk] = True
                g[nxt] = gather(nxt, bufs[nxt % _NBUF])
        for c in range(n_chunks):
            if not waited[c]:
                o[c].wait()

    return gather_kernel


def kernel(x, embs):
    rows, row_len = x.shape
    V, D = embs.shape
    out = _build_gather(rows, row_len, V, D)(x.astype(jnp.int32), embs)
    return out.reshape(rows, row_len, D)


# speedup vs baseline: 1.0124x; 1.0124x over previous
import functools

import jax
import jax.numpy as jnp
from jax import lax
from jax.experimental import pallas as pl
from jax.experimental.pallas import tpu as pltpu
from jax.experimental.pallas import tpu_sc as plsc

_NC = 2
_NS = 16
_CHUNK = 32
_NBUF = 5


@functools.lru_cache(maxsize=None)
def _build_gather(rows, row_len, V, D):
    B = rows * row_len
    NW = _NC * _NS
    assert B % (8 * NW) == 0
    b_per_w = B // NW
    assert row_len % b_per_w == 0
    C = min(_CHUNK, b_per_w)
    n_chunks = b_per_w // C
    assert b_per_w % C == 0

    mesh = plsc.VectorSubcoreMesh(core_axis_name="c", subcore_axis_name="s")

    @functools.partial(
        pl.kernel,
        mesh=mesh,
        out_type=jax.ShapeDtypeStruct((B, D), jnp.float32),
        scratch_types=(
            [pltpu.VMEM((b_per_w,), jnp.int32)]
            + [pltpu.VMEM((C, D), jnp.float32) for _ in range(_NBUF)]
            + [pltpu.SemaphoreType.DMA, pltpu.SemaphoreType.DMA]
        ),
    )
    def gather_kernel(idx_hbm, table_hbm, out_hbm, idx_v, *rest):
        bufs = rest[:_NBUF]
        gsem, osem = rest[_NBUF], rest[_NBUF + 1]
        wid = lax.axis_index("c") * _NS + lax.axis_index("s")
        base = wid * b_per_w
        row_len = idx_hbm.shape[1]
        pltpu.sync_copy(
            idx_hbm.at[base // row_len, pl.ds(base % row_len, b_per_w)],
            idx_v)

        def gather(c, buf):
            return pltpu.async_copy(
                table_hbm.at[idx_v.at[pl.ds(c * C, C)]], buf, gsem)

        def put(c, buf):
            return pltpu.async_copy(
                buf, out_hbm.at[pl.ds(base + c * C, C)], osem)

        g = [None] * n_chunks
        o = [None] * n_chunks
        waited = [False] * n_chunks
        for c in range(min(_NBUF, n_chunks)):
            g[c] = gather(c, bufs[c % _NBUF])
        for c in range(n_chunks):
            buf = bufs[c % _NBUF]
            g[c].wait()
            o[c] = put(c, buf)
            nxt = c + _NBUF - 1
            if nxt < n_chunks and nxt >= _NBUF:
                free_chunk = nxt - _NBUF
                o[free_chunk].wait()
                waited[free_chunk] = True
                g[nxt] = gather(nxt, bufs[nxt % _NBUF])
        for c in range(n_chunks):
            if not waited[c]:
                o[c].wait()

    return gather_kernel


def kernel(x, embs):
    rows, row_len = x.shape
    V, D = embs.shape
    out = _build_gather(rows, row_len, V, D)(x.astype(jnp.int32), embs)
    return out.reshape(rows, row_len, D)
